# Initial kernel scaffold; baseline (speedup 1.0000x reference)
import jax
import jax.numpy as jnp
from jax.experimental import pallas as pl


def kernel(feat_H, feat_L, feat_O, feat_M, feat_N, cb_H, cb_L, cb_O, cb_M, cb_N):
    raise NotImplementedError("write your pallas kernel here")


# speedup vs baseline: 1.7258x; 1.7258x over previous
import functools

import jax
import jax.numpy as jnp
from jax.experimental import pallas as pl
from jax.experimental.pallas import tpu as pltpu

B, T, D, K = 16, 1024, 64, 1024
N = B * T
R = 1024
NT = N // R


def _vq_tile(x_ref, cbt_ref, cb_ref, q_ref, idx_ref, loss_ref, cn_ref):
    i = pl.program_id(0)

    @pl.when(i == 0)
    def _init():
        cb = cb_ref[...]
        cn_ref[0, :] = jnp.sum(cb * cb, axis=1)
        loss_ref[0, 0] = jnp.float32(0.0)

    x = x_ref[...]
    xn = jnp.sum(x * x, axis=1, keepdims=True)
    dot = jnp.dot(x, cbt_ref[...], preferred_element_type=jnp.float32)
    dist = (xn + cn_ref[0, :][None, :]) - 2.0 * dot
    minv = jnp.min(dist, axis=1, keepdims=True)
    iota = jax.lax.broadcasted_iota(jnp.int32, (R, K), 1)
    cand = jnp.where(dist == minv, iota, K)
    idx = jnp.min(cand, axis=1)
    onehot = (iota == idx[:, None]).astype(jnp.float32)
    q_ref[...] = jnp.dot(onehot, cb_ref[...], preferred_element_type=jnp.float32)
    idx_ref[0, 0, :] = idx
    loss_ref[0, 0] += jnp.sum(minv)


@functools.partial(jax.jit, static_argnames=())
def _vq_one(x, cb):
    xf = x.reshape(N, D)
    cbt = cb.T
    q, idx3, loss_sum = pl.pallas_call(
        _vq_tile,
        grid=(NT,),
        in_specs=[
            pl.BlockSpec((R, D), lambda i: (i, 0)),
            pl.BlockSpec((D, K), lambda i: (0, 0)),
            pl.BlockSpec((K, D), lambda i: (0, 0)),
        ],
        out_specs=[
            pl.BlockSpec((R, D), lambda i: (i, 0)),
            pl.BlockSpec((1, 1, R), lambda i: (i, 0, 0)),
            pl.BlockSpec(memory_space=pltpu.SMEM, block_shape=(1, 1),
                         index_map=lambda i: (0, 0)),
        ],
        out_shape=[
            jax.ShapeDtypeStruct((N, D), jnp.float32),
            jax.ShapeDtypeStruct((NT, 1, R), jnp.int32),
            jax.ShapeDtypeStruct((1, 1), jnp.float32),
        ],
        scratch_shapes=[pltpu.VMEM((1, K), jnp.float32)],
    )(xf, cbt, cb)
    return (q.reshape(B, T, D), idx3.reshape(B, T),
            loss_sum[0, 0] / jnp.float32(N * D))


def kernel(feat_H, feat_L, feat_O, feat_M, feat_N, cb_H, cb_L, cb_O, cb_M, cb_N):
    feats = [feat_H, feat_L, feat_O, feat_M, feat_N]
    cbs = [cb_H, cb_L, cb_O, cb_M, cb_N]
    quantized, indices = [], []
    total_loss = jnp.float32(0.0)
    for f, cb in zip(feats, cbs):
        q, idx, loss = _vq_one(f, cb)
        quantized.append(q)
        indices.append(idx)
        total_loss = total_loss + loss
    return (tuple(quantized), tuple(indices), total_loss)


# speedup vs baseline: 1.8244x; 1.0571x over previous
import functools

import jax
import jax.numpy as jnp
from jax.experimental import pallas as pl
from jax.experimental.pallas import tpu as pltpu

B, T, D, K = 16, 1024, 64, 1024
N = B * T
R = 1024
NT = N // R


def _vq_tile(x_ref, cbt2_ref, cb_ref, q_ref, idx_ref, loss_ref):
    i = pl.program_id(0)

    @pl.when(i == 0)
    def _init():
        loss_ref[0, 0] = jnp.float32(0.0)

    x = x_ref[...]
    xn = jnp.sum(x * x, axis=1, keepdims=True)
    cbt2 = cbt2_ref[...]
    cn = 0.25 * jnp.sum(cbt2 * cbt2, axis=0, keepdims=True)
    dot2 = jnp.dot(x, cbt2, preferred_element_type=jnp.float32)
    dist = (xn + cn) - dot2
    minv = jnp.min(dist, axis=1, keepdims=True)
    iota = jax.lax.broadcasted_iota(jnp.int32, (1, K), 1).astype(jnp.float32)
    candf = jnp.where(dist == minv, iota, jnp.float32(K))
    idxf = jnp.min(candf, axis=1, keepdims=True)
    onehot = (iota == idxf).astype(jnp.float32)
    q_ref[...] = jnp.dot(onehot, cb_ref[...], preferred_element_type=jnp.float32)
    idx_ref[0, 0, :] = idxf[:, 0].astype(jnp.int32)
    loss_ref[0, 0] += jnp.sum(minv)


@functools.partial(jax.jit, static_argnames=())
def _vq_one(x, cb):
    xf = x.reshape(N, D)
    cbt2 = 2.0 * cb.T
    q, idx3, loss_sum = pl.pallas_call(
        _vq_tile,
        grid=(NT,),
        in_specs=[
            pl.BlockSpec((R, D), lambda i: (i, 0)),
            pl.BlockSpec((D, K), lambda i: (0, 0)),
            pl.BlockSpec((K, D), lambda i: (0, 0)),
        ],
        out_specs=[
            pl.BlockSpec((R, D), lambda i: (i, 0)),
            pl.BlockSpec((1, 1, R), lambda i: (i, 0, 0)),
            pl.BlockSpec(memory_space=pltpu.SMEM, block_shape=(1, 1),
                         index_map=lambda i: (0, 0)),
        ],
        out_shape=[
            jax.ShapeDtypeStruct((N, D), jnp.float32),
            jax.ShapeDtypeStruct((NT, 1, R), jnp.int32),
            jax.ShapeDtypeStruct((1, 1), jnp.float32),
        ],
    )(xf, cbt2, cb)
    return (q.reshape(B, T, D), idx3.reshape(B, T),
            loss_sum[0, 0] / jnp.float32(N * D))


def kernel(feat_H, feat_L, feat_O, feat_M, feat_N, cb_H, cb_L, cb_O, cb_M, cb_N):
    feats = [feat_H, feat_L, feat_O, feat_M, feat_N]
    cbs = [cb_H, cb_L, cb_O, cb_M, cb_N]
    quantized, indices = [], []
    total_loss = jnp.float32(0.0)
    for f, cb in zip(feats, cbs):
        q, idx, loss = _vq_one(f, cb)
        quantized.append(q)
        indices.append(idx)
        total_loss = total_loss + loss
    return (tuple(quantized), tuple(indices), total_loss)
